# Initial kernel scaffold; baseline (speedup 1.0000x reference)
import jax
import jax.numpy as jnp
from jax.experimental import pallas as pl


def kernel(x, edge_index_rel0, edge_index_rel1, W1_0, b1_0, W1_1, b1_1, W2_0, b2_0, W2_1, b2_1, W_ih_f, W_hh_f, b_ih_f, b_hh_f, W_ih_r, W_hh_r, b_ih_r, b_hh_r):
    raise NotImplementedError("write your pallas kernel here")


# speedup vs baseline: 4.6612x; 4.6612x over previous
import functools

import jax
import jax.numpy as jnp
from jax import lax
from jax.experimental import pallas as pl
from jax.experimental.pallas import tpu as pltpu
from jax.experimental.pallas import tpu_sc as plsc

N = 10000
NP = 10240
D_IN = 128
D_HID = 128
D_OUT = 64
E = 160000
H = D_OUT // 2

CH = 128
NCHUNK = E // CH
NSUB = 16
ITERS = -(-NCHUNK // NSUB)
RSTRIPE = NP // NSUB

_mesh = plsc.VectorSubcoreMesh(core_axis_name="c", subcore_axis_name="s",
                               num_cores=2, num_subcores=NSUB)


def _deg_body(edges, ones_hbm, zeros16, degs_out, acc0, acc1, ones_v, idx_v):
    c = lax.axis_index("c")
    sid = lax.axis_index("s")
    r0 = sid * RSTRIPE
    pltpu.sync_copy(zeros16.at[pl.ds(r0, RSTRIPE)], acc0.at[pl.ds(r0, RSTRIPE)])
    pltpu.sync_copy(zeros16.at[pl.ds(r0, RSTRIPE)], acc1.at[pl.ds(r0, RSTRIPE)])
    pltpu.sync_copy(ones_hbm, ones_v)
    plsc.subcore_barrier()

    for j, acc in ((0, acc0), (1, acc1)):
        off = (2 * c + j) * E

        def chunk(i, carry, acc=acc, off=off):
            cidx = i * NSUB + sid

            @pl.when(cidx < NCHUNK)
            def _():
                pltpu.sync_copy(edges.at[pl.ds(off + cidx * CH, CH)], idx_v)
                pltpu.sync_copy(ones_v, acc.at[idx_v], add=True)

            return carry

        lax.fori_loop(0, ITERS, chunk, 0)

    plsc.subcore_barrier()
    pltpu.sync_copy(acc0.at[pl.ds(r0, RSTRIPE)],
                    degs_out.at[pl.ds((2 * c) * NP + r0, RSTRIPE)])
    pltpu.sync_copy(acc1.at[pl.ds(r0, RSTRIPE)],
                    degs_out.at[pl.ds((2 * c + 1) * NP + r0, RSTRIPE)])


_deg_kernel = pl.kernel(
    _deg_body,
    out_type=jax.ShapeDtypeStruct((4 * NP, 16), jnp.float32),
    mesh=_mesh,
    compiler_params=pltpu.CompilerParams(use_tc_tiling_on_sc=False),
    scratch_types=[
        pltpu.VMEM_SHARED((NP, 16), jnp.float32),
        pltpu.VMEM_SHARED((NP, 16), jnp.float32),
        pltpu.VMEM((CH, 16), jnp.float32),
        pltpu.VMEM((CH,), jnp.int32),
    ],
)


def _agg_body(y_flat, edges, zerosd, agg_out, acc, rows, idx_s, idx_d, sem):
    c = lax.axis_index("c")
    sid = lax.axis_index("s")
    r0 = sid * RSTRIPE
    pltpu.sync_copy(zerosd.at[pl.ds(r0, RSTRIPE)], acc.at[pl.ds(r0, RSTRIPE)])
    plsc.subcore_barrier()

    src_off = (2 * c) * E
    dst_off = (2 * c + 1) * E
    row_off = c * NP

    def chunk(i, carry):
        cidx = i * NSUB + sid

        @pl.when(cidx < NCHUNK)
        def _():
            base = cidx * CH
            pltpu.sync_copy(edges.at[pl.ds(src_off + base, CH)], idx_s)
            pltpu.sync_copy(edges.at[pl.ds(dst_off + base, CH)], idx_d)
            for k in range(CH // 16):
                sl = pl.ds(k * 16, 16)
                idx_s[sl] = idx_s[sl] + row_off
            pltpu.async_copy(y_flat.at[idx_s], rows, sem).wait()
            pltpu.sync_copy(rows, acc.at[idx_d], add=True)

        return carry

    lax.fori_loop(0, ITERS, chunk, 0)

    plsc.subcore_barrier()
    pltpu.sync_copy(acc.at[pl.ds(r0, RSTRIPE)],
                    agg_out.at[pl.ds(row_off + r0, RSTRIPE)])


def _make_agg_kernel(d):
    return pl.kernel(
        _agg_body,
        out_type=jax.ShapeDtypeStruct((2 * NP, d), jnp.float32),
        mesh=_mesh,
        compiler_params=pltpu.CompilerParams(use_tc_tiling_on_sc=(d == 128)),
        scratch_types=[
            pltpu.VMEM_SHARED((NP, d), jnp.float32),
            pltpu.VMEM((CH, d), jnp.float32),
            pltpu.VMEM((CH,), jnp.int32),
            pltpu.VMEM((CH,), jnp.int32),
            pltpu.SemaphoreType.DMA,
        ],
    )


_agg128 = _make_agg_kernel(D_HID)
_agg64 = _make_agg_kernel(D_OUT)


BM = 1024
GRID_M = NP // BM


def _scale(deg_blk):
    return lax.rsqrt(jnp.maximum(deg_blk[:, :1], 1.0))


def _tc1_body(x_ref, w_ref, dout_ref, y_ref):
    y = jnp.dot(x_ref[...], w_ref[...], preferred_element_type=jnp.float32)
    y_ref[...] = y * _scale(dout_ref[...])


_tc1_specs = [
    pl.BlockSpec((BM, D_IN), lambda r, i: (i, 0)),
    pl.BlockSpec((None, D_IN, D_HID), lambda r, i: (r, 0, 0)),
    pl.BlockSpec((None, BM, 16), lambda r, i: (2 * r, i, 0)),
]
_tc1_ospec = pl.BlockSpec((None, BM, D_HID), lambda r, i: (r, i, 0))
_tc1 = pl.pallas_call(
    _tc1_body,
    grid=(2, GRID_M),
    in_specs=_tc1_specs,
    out_specs=_tc1_ospec,
    out_shape=jax.ShapeDtypeStruct((2, NP, D_HID), jnp.float32),
)


def _tc2_body(a0_ref, a1_ref, din0_ref, din1_ref, b1_ref, w2_ref, dout_ref,
              y2_ref):
    h = 0.5 * (a0_ref[...] * _scale(din0_ref[...])
               + a1_ref[...] * _scale(din1_ref[...])
               + b1_ref[0] + b1_ref[1])
    h = jnp.maximum(h, 0.0)
    y2 = jnp.dot(h, w2_ref[...], preferred_element_type=jnp.float32)
    y2_ref[...] = y2 * _scale(dout_ref[...])


_tc2_specs = [
    pl.BlockSpec((None, BM, D_HID), lambda r, i: (0, i, 0)),
    pl.BlockSpec((None, BM, D_HID), lambda r, i: (1, i, 0)),
    pl.BlockSpec((None, BM, 16), lambda r, i: (1, i, 0)),
    pl.BlockSpec((None, BM, 16), lambda r, i: (3, i, 0)),
    pl.BlockSpec((2, 1, D_HID), lambda r, i: (0, 0, 0)),
    pl.BlockSpec((None, D_HID, D_OUT), lambda r, i: (r, 0, 0)),
    pl.BlockSpec((None, BM, 16), lambda r, i: (2 * r, i, 0)),
]
_tc2_ospec = pl.BlockSpec((None, BM, D_OUT), lambda r, i: (r, i, 0))
_tc2 = pl.pallas_call(
    _tc2_body,
    grid=(2, GRID_M),
    in_specs=_tc2_specs,
    out_specs=_tc2_ospec,
    out_shape=jax.ShapeDtypeStruct((2, NP, D_OUT), jnp.float32),
)


def _tc3_body(a0_ref, a1_ref, din0_ref, din1_ref, b2_ref, wcat_ref, bcat_ref,
              out_ref):
    h2 = 0.5 * (a0_ref[...] * _scale(din0_ref[...])
                + a1_ref[...] * _scale(din1_ref[...])
                + b2_ref[0] + b2_ref[1])
    g = jnp.dot(h2, wcat_ref[...], preferred_element_type=jnp.float32)
    g = g + bcat_ref[...]
    sig = jax.nn.sigmoid
    hf = sig(g[:, 3 * H:4 * H]) * jnp.tanh(
        sig(g[:, 0:H]) * jnp.tanh(g[:, 2 * H:3 * H]))
    hb = sig(g[:, 7 * H:8 * H]) * jnp.tanh(
        sig(g[:, 4 * H:5 * H]) * jnp.tanh(g[:, 6 * H:7 * H]))
    out_ref[...] = jnp.concatenate([hf, hb], axis=1)


_tc3_specs = [
    pl.BlockSpec((None, BM, D_OUT), lambda i: (0, i, 0)),
    pl.BlockSpec((None, BM, D_OUT), lambda i: (1, i, 0)),
    pl.BlockSpec((None, BM, 16), lambda i: (1, i, 0)),
    pl.BlockSpec((None, BM, 16), lambda i: (3, i, 0)),
    pl.BlockSpec((2, 1, D_OUT), lambda i: (0, 0, 0)),
    pl.BlockSpec((D_OUT, 8 * H), lambda i: (0, 0)),
    pl.BlockSpec((1, 8 * H), lambda i: (0, 0)),
]
_tc3_ospec = pl.BlockSpec((BM, D_OUT), lambda i: (i, 0))
_tc3 = pl.pallas_call(
    _tc3_body,
    grid=(GRID_M,),
    in_specs=_tc3_specs,
    out_specs=_tc3_ospec,
    out_shape=jax.ShapeDtypeStruct((NP, D_OUT), jnp.float32),
)


def kernel(x, edge_index_rel0, edge_index_rel1, W1_0, b1_0, W1_1, b1_1,
           W2_0, b2_0, W2_1, b2_1, W_ih_f, W_hh_f, b_ih_f, b_hh_f,
           W_ih_r, W_hh_r, b_ih_r, b_hh_r):
    edges = jnp.concatenate([
        edge_index_rel0[0], edge_index_rel0[1],
        edge_index_rel1[0], edge_index_rel1[1],
    ])
    ones16 = jnp.ones((CH, 16), jnp.float32)
    zeros16 = jnp.zeros((NP, 16), jnp.float32)
    degs = _deg_kernel(edges, ones16, zeros16).reshape(4, NP, 16)

    x_pad = jnp.pad(x, ((0, NP - N), (0, 0)))
    w1s = jnp.stack([W1_0, W1_1])
    y1 = _tc1(x_pad, w1s, degs)

    zeros128 = jnp.zeros((NP, D_HID), jnp.float32)
    agg1 = _agg128(y1.reshape(2 * NP, D_HID), edges, zeros128)
    agg1 = agg1.reshape(2, NP, D_HID)

    b1s = jnp.stack([b1_0, b1_1]).reshape(2, 1, D_HID)
    w2s = jnp.stack([W2_0, W2_1])
    y2 = _tc2(agg1, agg1, degs, degs, b1s, w2s, degs)

    zeros64 = jnp.zeros((NP, D_OUT), jnp.float32)
    agg2 = _agg64(y2.reshape(2 * NP, D_OUT), edges, zeros64)
    agg2 = agg2.reshape(2, NP, D_OUT)

    b2s = jnp.stack([b2_0, b2_1]).reshape(2, 1, D_OUT)
    wcat = jnp.concatenate([W_ih_f, W_ih_r], axis=0).T
    bcat = (jnp.concatenate([b_ih_f + b_hh_f, b_ih_r + b_hh_r])
            .reshape(1, 8 * H))
    out = _tc3(agg2, agg2, degs, degs, b2s, wcat, bcat)
    return out[:N]


# speedup vs baseline: 6.2242x; 1.3353x over previous
import functools

import jax
import jax.numpy as jnp
from jax import lax
from jax.experimental import pallas as pl
from jax.experimental.pallas import tpu as pltpu
from jax.experimental.pallas import tpu_sc as plsc

N = 10000
NP = 10240
D_IN = 128
D_HID = 128
D_OUT = 64
E = 160000
H = D_OUT // 2

CH = 128
NCHUNK = E // CH
NSUB = 16
ITERS = -(-NCHUNK // NSUB)
RSTRIPE = NP // NSUB

_mesh = plsc.VectorSubcoreMesh(core_axis_name="c", subcore_axis_name="s",
                               num_cores=2, num_subcores=NSUB)


def _deg_body(edges, ones_hbm, zeros16, degs_out, acc0, acc1, ones_v, idx_v):
    c = lax.axis_index("c")
    sid = lax.axis_index("s")
    r0 = sid * RSTRIPE
    pltpu.sync_copy(zeros16.at[pl.ds(r0, RSTRIPE)], acc0.at[pl.ds(r0, RSTRIPE)])
    pltpu.sync_copy(zeros16.at[pl.ds(r0, RSTRIPE)], acc1.at[pl.ds(r0, RSTRIPE)])
    pltpu.sync_copy(ones_hbm, ones_v)
    plsc.subcore_barrier()

    for j, acc in ((0, acc0), (1, acc1)):
        off = (2 * c + j) * E

        def chunk(i, carry, acc=acc, off=off):
            cidx = i * NSUB + sid

            @pl.when(cidx < NCHUNK)
            def _():
                pltpu.sync_copy(edges.at[pl.ds(off + cidx * CH, CH)], idx_v)
                pltpu.sync_copy(ones_v, acc.at[idx_v], add=True)

            return carry

        lax.fori_loop(0, ITERS, chunk, 0)

    plsc.subcore_barrier()
    pltpu.sync_copy(acc0.at[pl.ds(r0, RSTRIPE)],
                    degs_out.at[pl.ds((2 * c) * NP + r0, RSTRIPE)])
    pltpu.sync_copy(acc1.at[pl.ds(r0, RSTRIPE)],
                    degs_out.at[pl.ds((2 * c + 1) * NP + r0, RSTRIPE)])


_deg_kernel = pl.kernel(
    _deg_body,
    out_type=jax.ShapeDtypeStruct((4 * NP, 16), jnp.float32),
    mesh=_mesh,
    compiler_params=pltpu.CompilerParams(use_tc_tiling_on_sc=False),
    scratch_types=[
        pltpu.VMEM_SHARED((NP, 16), jnp.float32),
        pltpu.VMEM_SHARED((NP, 16), jnp.float32),
        pltpu.VMEM((CH, 16), jnp.float32),
        pltpu.VMEM((CH,), jnp.int32),
    ],
)


def _agg_body(y_flat, edges, zerosd, agg_out, acc,
              rows0, rows1, idx_s0, idx_s1, idx_d0, idx_d1, sem0, sem1):
    c = lax.axis_index("c")
    sid = lax.axis_index("s")
    r0 = sid * RSTRIPE
    pltpu.sync_copy(zerosd.at[pl.ds(r0, RSTRIPE)], acc.at[pl.ds(r0, RSTRIPE)])
    plsc.subcore_barrier()

    src_off = (2 * c) * E
    dst_off = (2 * c + 1) * E
    row_off = c * NP

    bufs = ((rows0, idx_s0, idx_d0, sem0), (rows1, idx_s1, idx_d1, sem1))

    def load_and_gather(k, buf):
        rows, idx_s, idx_d, sem = buf
        cidx = k * NSUB + sid

        @pl.when(cidx < NCHUNK)
        def _():
            base = cidx * CH
            pltpu.sync_copy(edges.at[pl.ds(src_off + base, CH)], idx_s)
            pltpu.sync_copy(edges.at[pl.ds(dst_off + base, CH)], idx_d)
            for kk in range(CH // 16):
                sl = pl.ds(kk * 16, 16)
                idx_s[sl] = idx_s[sl] + row_off
            pltpu.async_copy(y_flat.at[idx_s], rows, sem)

    def drain_and_scatter(k, buf):
        rows, idx_s, idx_d, sem = buf
        cidx = k * NSUB + sid

        @pl.when(cidx < NCHUNK)
        def _():
            pltpu.make_async_copy(y_flat.at[idx_s], rows, sem).wait()
            pltpu.sync_copy(rows, acc.at[idx_d], add=True)

    load_and_gather(0, bufs[0])

    def pair(j, carry):
        a = 2 * j
        load_and_gather(a + 1, bufs[1])
        drain_and_scatter(a, bufs[0])
        load_and_gather(a + 2, bufs[0])
        drain_and_scatter(a + 1, bufs[1])
        return carry

    lax.fori_loop(0, ITERS // 2, pair, 0)
    drain_and_scatter(ITERS - 1, bufs[0])

    plsc.subcore_barrier()
    pltpu.sync_copy(acc.at[pl.ds(r0, RSTRIPE)],
                    agg_out.at[pl.ds(row_off + r0, RSTRIPE)])


def _make_agg_kernel(d):
    return pl.kernel(
        _agg_body,
        out_type=jax.ShapeDtypeStruct((2 * NP, d), jnp.float32),
        mesh=_mesh,
        compiler_params=pltpu.CompilerParams(use_tc_tiling_on_sc=(d == 128)),
        scratch_types=[
            pltpu.VMEM_SHARED((NP, d), jnp.float32),
            pltpu.VMEM((CH, d), jnp.float32),
            pltpu.VMEM((CH, d), jnp.float32),
            pltpu.VMEM((CH,), jnp.int32),
            pltpu.VMEM((CH,), jnp.int32),
            pltpu.VMEM((CH,), jnp.int32),
            pltpu.VMEM((CH,), jnp.int32),
            pltpu.SemaphoreType.DMA,
            pltpu.SemaphoreType.DMA,
        ],
    )


_agg128 = _make_agg_kernel(D_HID)
_agg64 = _make_agg_kernel(D_OUT)


BM = 1024
GRID_M = NP // BM


def _scale(deg_blk):
    return lax.rsqrt(jnp.maximum(deg_blk[:, :1], 1.0))


def _tc1_body(x_ref, w_ref, dout_ref, y_ref):
    y = jnp.dot(x_ref[...], w_ref[...], preferred_element_type=jnp.float32)
    y_ref[...] = y * _scale(dout_ref[...])


_tc1_specs = [
    pl.BlockSpec((BM, D_IN), lambda r, i: (i, 0)),
    pl.BlockSpec((None, D_IN, D_HID), lambda r, i: (r, 0, 0)),
    pl.BlockSpec((None, BM, 16), lambda r, i: (2 * r, i, 0)),
]
_tc1_ospec = pl.BlockSpec((None, BM, D_HID), lambda r, i: (r, i, 0))
_tc1 = pl.pallas_call(
    _tc1_body,
    grid=(2, GRID_M),
    in_specs=_tc1_specs,
    out_specs=_tc1_ospec,
    out_shape=jax.ShapeDtypeStruct((2, NP, D_HID), jnp.float32),
)


def _tc2_body(a0_ref, a1_ref, din0_ref, din1_ref, b1_ref, w2_ref, dout_ref,
              y2_ref):
    h = 0.5 * (a0_ref[...] * _scale(din0_ref[...])
               + a1_ref[...] * _scale(din1_ref[...])
               + b1_ref[0] + b1_ref[1])
    h = jnp.maximum(h, 0.0)
    y2 = jnp.dot(h, w2_ref[...], preferred_element_type=jnp.float32)
    y2_ref[...] = y2 * _scale(dout_ref[...])


_tc2_specs = [
    pl.BlockSpec((None, BM, D_HID), lambda r, i: (0, i, 0)),
    pl.BlockSpec((None, BM, D_HID), lambda r, i: (1, i, 0)),
    pl.BlockSpec((None, BM, 16), lambda r, i: (1, i, 0)),
    pl.BlockSpec((None, BM, 16), lambda r, i: (3, i, 0)),
    pl.BlockSpec((2, 1, D_HID), lambda r, i: (0, 0, 0)),
    pl.BlockSpec((None, D_HID, D_OUT), lambda r, i: (r, 0, 0)),
    pl.BlockSpec((None, BM, 16), lambda r, i: (2 * r, i, 0)),
]
_tc2_ospec = pl.BlockSpec((None, BM, D_OUT), lambda r, i: (r, i, 0))
_tc2 = pl.pallas_call(
    _tc2_body,
    grid=(2, GRID_M),
    in_specs=_tc2_specs,
    out_specs=_tc2_ospec,
    out_shape=jax.ShapeDtypeStruct((2, NP, D_OUT), jnp.float32),
)


def _tc3_body(a0_ref, a1_ref, din0_ref, din1_ref, b2_ref, wcat_ref, bcat_ref,
              out_ref):
    h2 = 0.5 * (a0_ref[...] * _scale(din0_ref[...])
                + a1_ref[...] * _scale(din1_ref[...])
                + b2_ref[0] + b2_ref[1])
    g = jnp.dot(h2, wcat_ref[...], preferred_element_type=jnp.float32)
    g = g + bcat_ref[...]
    sig = jax.nn.sigmoid
    hf = sig(g[:, 3 * H:4 * H]) * jnp.tanh(
        sig(g[:, 0:H]) * jnp.tanh(g[:, 2 * H:3 * H]))
    hb = sig(g[:, 7 * H:8 * H]) * jnp.tanh(
        sig(g[:, 4 * H:5 * H]) * jnp.tanh(g[:, 6 * H:7 * H]))
    out_ref[...] = jnp.concatenate([hf, hb], axis=1)


_tc3_specs = [
    pl.BlockSpec((None, BM, D_OUT), lambda i: (0, i, 0)),
    pl.BlockSpec((None, BM, D_OUT), lambda i: (1, i, 0)),
    pl.BlockSpec((None, BM, 16), lambda i: (1, i, 0)),
    pl.BlockSpec((None, BM, 16), lambda i: (3, i, 0)),
    pl.BlockSpec((2, 1, D_OUT), lambda i: (0, 0, 0)),
    pl.BlockSpec((D_OUT, 8 * H), lambda i: (0, 0)),
    pl.BlockSpec((1, 8 * H), lambda i: (0, 0)),
]
_tc3_ospec = pl.BlockSpec((BM, D_OUT), lambda i: (i, 0))
_tc3 = pl.pallas_call(
    _tc3_body,
    grid=(GRID_M,),
    in_specs=_tc3_specs,
    out_specs=_tc3_ospec,
    out_shape=jax.ShapeDtypeStruct((NP, D_OUT), jnp.float32),
)


def kernel(x, edge_index_rel0, edge_index_rel1, W1_0, b1_0, W1_1, b1_1,
           W2_0, b2_0, W2_1, b2_1, W_ih_f, W_hh_f, b_ih_f, b_hh_f,
           W_ih_r, W_hh_r, b_ih_r, b_hh_r):
    edges = jnp.concatenate([
        edge_index_rel0[0], edge_index_rel0[1],
        edge_index_rel1[0], edge_index_rel1[1],
    ])
    ones16 = jnp.ones((CH, 16), jnp.float32)
    zeros16 = jnp.zeros((NP, 16), jnp.float32)
    degs = _deg_kernel(edges, ones16, zeros16).reshape(4, NP, 16)

    x_pad = jnp.pad(x, ((0, NP - N), (0, 0)))
    w1s = jnp.stack([W1_0, W1_1])
    y1 = _tc1(x_pad, w1s, degs)

    zeros128 = jnp.zeros((NP, D_HID), jnp.float32)
    agg1 = _agg128(y1.reshape(2 * NP, D_HID), edges, zeros128)
    agg1 = agg1.reshape(2, NP, D_HID)

    b1s = jnp.stack([b1_0, b1_1]).reshape(2, 1, D_HID)
    w2s = jnp.stack([W2_0, W2_1])
    y2 = _tc2(agg1, agg1, degs, degs, b1s, w2s, degs)

    zeros64 = jnp.zeros((NP, D_OUT), jnp.float32)
    agg2 = _agg64(y2.reshape(2 * NP, D_OUT), edges, zeros64)
    agg2 = agg2.reshape(2, NP, D_OUT)

    b2s = jnp.stack([b2_0, b2_1]).reshape(2, 1, D_OUT)
    wcat = jnp.concatenate([W_ih_f, W_ih_r], axis=0).T
    bcat = (jnp.concatenate([b_ih_f + b_hh_f, b_ih_r + b_hh_r])
            .reshape(1, 8 * H))
    out = _tc3(agg2, agg2, degs, degs, b2s, wcat, bcat)
    return out[:N]


# speedup vs baseline: 7.1526x; 1.1492x over previous
import functools

import jax
import jax.numpy as jnp
from jax import lax
from jax.experimental import pallas as pl
from jax.experimental.pallas import tpu as pltpu
from jax.experimental.pallas import tpu_sc as plsc

N = 10000
NP = 10240
D_IN = 128
D_HID = 128
D_OUT = 64
E = 160000
H = D_OUT // 2

CH = 128
NCHUNK = E // CH
NSUB = 16
ITERS = -(-NCHUNK // NSUB)
RSTRIPE = NP // NSUB

_mesh = plsc.VectorSubcoreMesh(core_axis_name="c", subcore_axis_name="s",
                               num_cores=2, num_subcores=NSUB)


GROUP = 8
CPT = -(-NCHUNK // NSUB)
NGRP = -(-CPT // GROUP)


def _deg_body(edges2d, ones_hbm, zeros16, degs_out,
              acc0, acc1, ones_v, idx_s, idx_d, sem_a, sem_b):
    c = lax.axis_index("c")
    sid = lax.axis_index("s")
    r0 = sid * RSTRIPE
    pltpu.sync_copy(zeros16.at[pl.ds(r0, RSTRIPE)], acc0.at[pl.ds(r0, RSTRIPE)])
    pltpu.sync_copy(zeros16.at[pl.ds(r0, RSTRIPE)], acc1.at[pl.ds(r0, RSTRIPE)])
    pltpu.sync_copy(ones_hbm, ones_v)
    plsc.subcore_barrier()

    lo = sid * CPT
    srow = (2 * c) * NCHUNK
    drow = (2 * c + 1) * NCHUNK

    def grp(g, carry):
        base = lo + g * GROUP

        @pl.when(base < NCHUNK)
        def _():
            pltpu.sync_copy(edges2d.at[pl.ds(srow + base, GROUP)], idx_s)
            pltpu.sync_copy(edges2d.at[pl.ds(drow + base, GROUP)], idx_d)
            for j in range(GROUP):
                ok = ((g * GROUP + j) < CPT) & ((base + j) < NCHUNK)

                @pl.when(ok)
                def _(j=j):
                    pltpu.async_copy(ones_v, acc0.at[idx_s.at[j]], sem_a,
                                     add=True)
                    pltpu.async_copy(ones_v, acc1.at[idx_d.at[j]], sem_b,
                                     add=True)
            for j in range(GROUP):
                ok = ((g * GROUP + j) < CPT) & ((base + j) < NCHUNK)

                @pl.when(ok)
                def _(j=j):
                    pltpu.make_async_copy(ones_v, acc0.at[idx_s.at[j]],
                                          sem_a).wait()
                    pltpu.make_async_copy(ones_v, acc1.at[idx_d.at[j]],
                                          sem_b).wait()

        return carry

    lax.fori_loop(0, NGRP, grp, 0)

    plsc.subcore_barrier()
    pltpu.sync_copy(acc0.at[pl.ds(r0, RSTRIPE)],
                    degs_out.at[pl.ds((2 * c) * NP + r0, RSTRIPE)])
    pltpu.sync_copy(acc1.at[pl.ds(r0, RSTRIPE)],
                    degs_out.at[pl.ds((2 * c + 1) * NP + r0, RSTRIPE)])


_deg_kernel = pl.kernel(
    _deg_body,
    out_type=jax.ShapeDtypeStruct((4 * NP, 16), jnp.float32),
    mesh=_mesh,
    compiler_params=pltpu.CompilerParams(use_tc_tiling_on_sc=False),
    scratch_types=[
        pltpu.VMEM_SHARED((NP, 16), jnp.float32),
        pltpu.VMEM_SHARED((NP, 16), jnp.float32),
        pltpu.VMEM((CH, 16), jnp.float32),
        pltpu.VMEM((GROUP, CH), jnp.int32),
        pltpu.VMEM((GROUP, CH), jnp.int32),
        pltpu.SemaphoreType.DMA,
        pltpu.SemaphoreType.DMA,
    ],
)


def _agg_body(y_flat, edges, zerosd, agg_out, acc,
              rows0, rows1, idx_s0, idx_s1, idx_d0, idx_d1, sem0, sem1):
    c = lax.axis_index("c")
    sid = lax.axis_index("s")
    r0 = sid * RSTRIPE
    pltpu.sync_copy(zerosd.at[pl.ds(r0, RSTRIPE)], acc.at[pl.ds(r0, RSTRIPE)])
    plsc.subcore_barrier()

    src_off = (2 * c) * E
    dst_off = (2 * c + 1) * E
    row_off = c * NP

    bufs = ((rows0, idx_s0, idx_d0, sem0), (rows1, idx_s1, idx_d1, sem1))

    def load_and_gather(k, buf):
        rows, idx_s, idx_d, sem = buf
        cidx = k * NSUB + sid

        @pl.when(cidx < NCHUNK)
        def _():
            base = cidx * CH
            pltpu.sync_copy(edges.at[pl.ds(src_off + base, CH)], idx_s)
            pltpu.sync_copy(edges.at[pl.ds(dst_off + base, CH)], idx_d)
            for kk in range(CH // 16):
                sl = pl.ds(kk * 16, 16)
                idx_s[sl] = idx_s[sl] + row_off
            pltpu.async_copy(y_flat.at[idx_s], rows, sem)

    def drain_and_scatter(k, buf):
        rows, idx_s, idx_d, sem = buf
        cidx = k * NSUB + sid

        @pl.when(cidx < NCHUNK)
        def _():
            pltpu.make_async_copy(y_flat.at[idx_s], rows, sem).wait()
            pltpu.sync_copy(rows, acc.at[idx_d], add=True)

    load_and_gather(0, bufs[0])

    def pair(j, carry):
        a = 2 * j
        load_and_gather(a + 1, bufs[1])
        drain_and_scatter(a, bufs[0])
        load_and_gather(a + 2, bufs[0])
        drain_and_scatter(a + 1, bufs[1])
        return carry

    lax.fori_loop(0, ITERS // 2, pair, 0)
    drain_and_scatter(ITERS - 1, bufs[0])

    plsc.subcore_barrier()
    pltpu.sync_copy(acc.at[pl.ds(r0, RSTRIPE)],
                    agg_out.at[pl.ds(row_off + r0, RSTRIPE)])


def _make_agg_kernel(d):
    return pl.kernel(
        _agg_body,
        out_type=jax.ShapeDtypeStruct((2 * NP, d), jnp.float32),
        mesh=_mesh,
        compiler_params=pltpu.CompilerParams(use_tc_tiling_on_sc=(d == 128)),
        scratch_types=[
            pltpu.VMEM_SHARED((NP, d), jnp.float32),
            pltpu.VMEM((CH, d), jnp.float32),
            pltpu.VMEM((CH, d), jnp.float32),
            pltpu.VMEM((CH,), jnp.int32),
            pltpu.VMEM((CH,), jnp.int32),
            pltpu.VMEM((CH,), jnp.int32),
            pltpu.VMEM((CH,), jnp.int32),
            pltpu.SemaphoreType.DMA,
            pltpu.SemaphoreType.DMA,
        ],
    )


_agg128 = _make_agg_kernel(D_HID)
_agg64 = _make_agg_kernel(D_OUT)


BM = 1024
GRID_M = NP // BM


def _scale(deg_blk):
    return lax.rsqrt(jnp.maximum(deg_blk[:, :1], 1.0))


def _tc1_body(x_ref, w_ref, dout_ref, y_ref):
    y = jnp.dot(x_ref[...], w_ref[...], preferred_element_type=jnp.float32)
    y_ref[...] = y * _scale(dout_ref[...])


_tc1_specs = [
    pl.BlockSpec((BM, D_IN), lambda r, i: (i, 0)),
    pl.BlockSpec((None, D_IN, D_HID), lambda r, i: (r, 0, 0)),
    pl.BlockSpec((None, BM, 16), lambda r, i: (2 * r, i, 0)),
]
_tc1_ospec = pl.BlockSpec((None, BM, D_HID), lambda r, i: (r, i, 0))
_tc1 = pl.pallas_call(
    _tc1_body,
    grid=(2, GRID_M),
    in_specs=_tc1_specs,
    out_specs=_tc1_ospec,
    out_shape=jax.ShapeDtypeStruct((2, NP, D_HID), jnp.float32),
)


def _tc2_body(a0_ref, a1_ref, din0_ref, din1_ref, b1_ref, w2_ref, dout_ref,
              y2_ref):
    h = 0.5 * (a0_ref[...] * _scale(din0_ref[...])
               + a1_ref[...] * _scale(din1_ref[...])
               + b1_ref[0] + b1_ref[1])
    h = jnp.maximum(h, 0.0)
    y2 = jnp.dot(h, w2_ref[...], preferred_element_type=jnp.float32)
    y2_ref[...] = y2 * _scale(dout_ref[...])


_tc2_specs = [
    pl.BlockSpec((None, BM, D_HID), lambda r, i: (0, i, 0)),
    pl.BlockSpec((None, BM, D_HID), lambda r, i: (1, i, 0)),
    pl.BlockSpec((None, BM, 16), lambda r, i: (1, i, 0)),
    pl.BlockSpec((None, BM, 16), lambda r, i: (3, i, 0)),
    pl.BlockSpec((2, 1, D_HID), lambda r, i: (0, 0, 0)),
    pl.BlockSpec((None, D_HID, D_OUT), lambda r, i: (r, 0, 0)),
    pl.BlockSpec((None, BM, 16), lambda r, i: (2 * r, i, 0)),
]
_tc2_ospec = pl.BlockSpec((None, BM, D_OUT), lambda r, i: (r, i, 0))
_tc2 = pl.pallas_call(
    _tc2_body,
    grid=(2, GRID_M),
    in_specs=_tc2_specs,
    out_specs=_tc2_ospec,
    out_shape=jax.ShapeDtypeStruct((2, NP, D_OUT), jnp.float32),
)


def _tc3_body(a0_ref, a1_ref, din0_ref, din1_ref, b2_ref, wcat_ref, bcat_ref,
              out_ref):
    h2 = 0.5 * (a0_ref[...] * _scale(din0_ref[...])
                + a1_ref[...] * _scale(din1_ref[...])
                + b2_ref[0] + b2_ref[1])
    g = jnp.dot(h2, wcat_ref[...], preferred_element_type=jnp.float32)
    g = g + bcat_ref[...]
    sig = jax.nn.sigmoid
    hf = sig(g[:, 3 * H:4 * H]) * jnp.tanh(
        sig(g[:, 0:H]) * jnp.tanh(g[:, 2 * H:3 * H]))
    hb = sig(g[:, 7 * H:8 * H]) * jnp.tanh(
        sig(g[:, 4 * H:5 * H]) * jnp.tanh(g[:, 6 * H:7 * H]))
    out_ref[...] = jnp.concatenate([hf, hb], axis=1)


_tc3_specs = [
    pl.BlockSpec((None, BM, D_OUT), lambda i: (0, i, 0)),
    pl.BlockSpec((None, BM, D_OUT), lambda i: (1, i, 0)),
    pl.BlockSpec((None, BM, 16), lambda i: (1, i, 0)),
    pl.BlockSpec((None, BM, 16), lambda i: (3, i, 0)),
    pl.BlockSpec((2, 1, D_OUT), lambda i: (0, 0, 0)),
    pl.BlockSpec((D_OUT, 8 * H), lambda i: (0, 0)),
    pl.BlockSpec((1, 8 * H), lambda i: (0, 0)),
]
_tc3_ospec = pl.BlockSpec((BM, D_OUT), lambda i: (i, 0))
_tc3 = pl.pallas_call(
    _tc3_body,
    grid=(GRID_M,),
    in_specs=_tc3_specs,
    out_specs=_tc3_ospec,
    out_shape=jax.ShapeDtypeStruct((NP, D_OUT), jnp.float32),
)


def kernel(x, edge_index_rel0, edge_index_rel1, W1_0, b1_0, W1_1, b1_1,
           W2_0, b2_0, W2_1, b2_1, W_ih_f, W_hh_f, b_ih_f, b_hh_f,
           W_ih_r, W_hh_r, b_ih_r, b_hh_r):
    edges = jnp.concatenate([
        edge_index_rel0[0], edge_index_rel0[1],
        edge_index_rel1[0], edge_index_rel1[1],
        jnp.zeros((GROUP * CH,), jnp.int32),
    ])
    edges2d = edges.reshape(-1, CH)
    ones16 = jnp.ones((CH, 16), jnp.float32)
    zeros16 = jnp.zeros((NP, 16), jnp.float32)
    degs = _deg_kernel(edges2d, ones16, zeros16).reshape(4, NP, 16)

    x_pad = jnp.pad(x, ((0, NP - N), (0, 0)))
    w1s = jnp.stack([W1_0, W1_1])
    y1 = _tc1(x_pad, w1s, degs)

    zeros128 = jnp.zeros((NP, D_HID), jnp.float32)
    agg1 = _agg128(y1.reshape(2 * NP, D_HID), edges, zeros128)
    agg1 = agg1.reshape(2, NP, D_HID)

    b1s = jnp.stack([b1_0, b1_1]).reshape(2, 1, D_HID)
    w2s = jnp.stack([W2_0, W2_1])
    y2 = _tc2(agg1, agg1, degs, degs, b1s, w2s, degs)

    zeros64 = jnp.zeros((NP, D_OUT), jnp.float32)
    agg2 = _agg64(y2.reshape(2 * NP, D_OUT), edges, zeros64)
    agg2 = agg2.reshape(2, NP, D_OUT)

    b2s = jnp.stack([b2_0, b2_1]).reshape(2, 1, D_OUT)
    wcat = jnp.concatenate([W_ih_f, W_ih_r], axis=0).T
    bcat = (jnp.concatenate([b_ih_f + b_hh_f, b_ih_r + b_hh_r])
            .reshape(1, 8 * H))
    out = _tc3(agg2, agg2, degs, degs, b2s, wcat, bcat)
    return out[:N]


# speedup vs baseline: 8.0611x; 1.1270x over previous
import functools

import jax
import jax.numpy as jnp
from jax import lax
from jax.experimental import pallas as pl
from jax.experimental.pallas import tpu as pltpu
from jax.experimental.pallas import tpu_sc as plsc

N = 10000
NP = 10240
D_IN = 128
D_HID = 128
D_OUT = 64
E = 160000
H = D_OUT // 2

CH = 128
NCHUNK = E // CH
NSUB = 16
ITERS = -(-NCHUNK // NSUB)
RSTRIPE = NP // NSUB

_mesh = plsc.VectorSubcoreMesh(core_axis_name="c", subcore_axis_name="s",
                               num_cores=2, num_subcores=NSUB)


GROUP = 8
CPT = (-(-NCHUNK // NSUB) + GROUP - 1) // GROUP * GROUP
NGRP = CPT // GROUP
NCHUNKP = -(-NCHUNK // GROUP) * GROUP


def _deg_body(edges2d, ones_hbm, zeros16, degs_out,
              acc0, acc1, ones_v, idx_s, idx_d, sem_a, sem_b):
    c = lax.axis_index("c")
    sid = lax.axis_index("s")
    r0 = sid * RSTRIPE
    pltpu.sync_copy(zeros16.at[pl.ds(r0, RSTRIPE)], acc0.at[pl.ds(r0, RSTRIPE)])
    pltpu.sync_copy(zeros16.at[pl.ds(r0, RSTRIPE)], acc1.at[pl.ds(r0, RSTRIPE)])
    pltpu.sync_copy(ones_hbm, ones_v)
    plsc.subcore_barrier()

    lo = sid * CPT
    srow = (2 * c) * NCHUNKP
    drow = (2 * c + 1) * NCHUNKP

    def grp(g, carry):
        base = lo + g * GROUP

        @pl.when(base < NCHUNK)
        def _():
            pltpu.sync_copy(edges2d.at[pl.ds(srow + base, GROUP)], idx_s)
            pltpu.sync_copy(edges2d.at[pl.ds(drow + base, GROUP)], idx_d)
            for j in range(GROUP):
                ok = ((g * GROUP + j) < CPT) & ((base + j) < NCHUNK)

                @pl.when(ok)
                def _(j=j):
                    pltpu.async_copy(ones_v, acc0.at[idx_s.at[j]], sem_a,
                                     add=True)
                    pltpu.async_copy(ones_v, acc1.at[idx_d.at[j]], sem_b,
                                     add=True)
            for j in range(GROUP):
                ok = ((g * GROUP + j) < CPT) & ((base + j) < NCHUNK)

                @pl.when(ok)
                def _(j=j):
                    pltpu.make_async_copy(ones_v, acc0.at[idx_s.at[j]],
                                          sem_a).wait()
                    pltpu.make_async_copy(ones_v, acc1.at[idx_d.at[j]],
                                          sem_b).wait()

        return carry

    lax.fori_loop(0, NGRP, grp, 0)

    plsc.subcore_barrier()
    pltpu.sync_copy(acc0.at[pl.ds(r0, RSTRIPE)],
                    degs_out.at[pl.ds((2 * c) * NP + r0, RSTRIPE)])
    pltpu.sync_copy(acc1.at[pl.ds(r0, RSTRIPE)],
                    degs_out.at[pl.ds((2 * c + 1) * NP + r0, RSTRIPE)])


_deg_kernel = pl.kernel(
    _deg_body,
    out_type=jax.ShapeDtypeStruct((4 * NP, 16), jnp.float32),
    mesh=_mesh,
    compiler_params=pltpu.CompilerParams(use_tc_tiling_on_sc=False),
    scratch_types=[
        pltpu.VMEM_SHARED((NP, 16), jnp.float32),
        pltpu.VMEM_SHARED((NP, 16), jnp.float32),
        pltpu.VMEM((CH, 16), jnp.float32),
        pltpu.VMEM((GROUP, CH), jnp.int32),
        pltpu.VMEM((GROUP, CH), jnp.int32),
        pltpu.SemaphoreType.DMA,
        pltpu.SemaphoreType.DMA,
    ],
)


def _agg_body(y_flat, edges2d, zerosd, agg_out, acc,
              rows0, rows1, idx_s, idx_d, sem0, sem1):
    c = lax.axis_index("c")
    sid = lax.axis_index("s")
    r0 = sid * RSTRIPE
    pltpu.sync_copy(zerosd.at[pl.ds(r0, RSTRIPE)], acc.at[pl.ds(r0, RSTRIPE)])
    plsc.subcore_barrier()

    srow = (2 * c) * NCHUNKP
    drow = (2 * c + 1) * NCHUNKP
    row_off = c * NP
    lo = sid * CPT
    rowbufs = (rows0, rows1)
    sems = (sem0, sem1)

    def grp(g, carry):
        base = lo + g * GROUP

        @pl.when(base < NCHUNK)
        def _():
            pltpu.sync_copy(edges2d.at[pl.ds(srow + base, GROUP)], idx_s)
            pltpu.sync_copy(edges2d.at[pl.ds(drow + base, GROUP)], idx_d)
            for jj in range(GROUP):
                for k in range(CH // 16):
                    sl = pl.ds(k * 16, 16)
                    idx_s[jj, sl] = idx_s[jj, sl] + row_off

            def ok(j):
                return ((g * GROUP + j) < CPT) & ((base + j) < NCHUNK)

            @pl.when(ok(0))
            def _():
                pltpu.async_copy(y_flat.at[idx_s.at[0]], rows0, sem0)

            for j in range(GROUP):
                rows, sem = rowbufs[j % 2], sems[j % 2]
                if j + 1 < GROUP:
                    nrows, nsem = rowbufs[(j + 1) % 2], sems[(j + 1) % 2]

                    @pl.when(ok(j + 1))
                    def _(j=j, nrows=nrows, nsem=nsem):
                        pltpu.async_copy(y_flat.at[idx_s.at[j + 1]], nrows,
                                         nsem)

                @pl.when(ok(j))
                def _(j=j, rows=rows, sem=sem):
                    pltpu.make_async_copy(y_flat.at[idx_s.at[j]], rows,
                                          sem).wait()
                    pltpu.sync_copy(rows, acc.at[idx_d.at[j]], add=True)

        return carry

    lax.fori_loop(0, NGRP, grp, 0)

    plsc.subcore_barrier()
    pltpu.sync_copy(acc.at[pl.ds(r0, RSTRIPE)],
                    agg_out.at[pl.ds(row_off + r0, RSTRIPE)])


def _make_agg_kernel(d):
    return pl.kernel(
        _agg_body,
        out_type=jax.ShapeDtypeStruct((2 * NP, d), jnp.float32),
        mesh=_mesh,
        compiler_params=pltpu.CompilerParams(use_tc_tiling_on_sc=(d == 128)),
        scratch_types=[
            pltpu.VMEM_SHARED((NP, d), jnp.float32),
            pltpu.VMEM((CH, d), jnp.float32),
            pltpu.VMEM((CH, d), jnp.float32),
            pltpu.VMEM((GROUP, CH), jnp.int32),
            pltpu.VMEM((GROUP, CH), jnp.int32),
            pltpu.SemaphoreType.DMA,
            pltpu.SemaphoreType.DMA,
        ],
    )


_agg128 = _make_agg_kernel(D_HID)
_agg64 = _make_agg_kernel(D_OUT)


BM = 1024
GRID_M = NP // BM


def _scale(deg_blk):
    return lax.rsqrt(jnp.maximum(deg_blk[:, :1], 1.0))


def _tc1_body(x_ref, w_ref, dout_ref, y_ref):
    y = jnp.dot(x_ref[...], w_ref[...], preferred_element_type=jnp.float32)
    y_ref[...] = y * _scale(dout_ref[...])


_tc1_specs = [
    pl.BlockSpec((BM, D_IN), lambda r, i: (i, 0)),
    pl.BlockSpec((None, D_IN, D_HID), lambda r, i: (r, 0, 0)),
    pl.BlockSpec((None, BM, 16), lambda r, i: (2 * r, i, 0)),
]
_tc1_ospec = pl.BlockSpec((None, BM, D_HID), lambda r, i: (r, i, 0))
_tc1 = pl.pallas_call(
    _tc1_body,
    grid=(2, GRID_M),
    in_specs=_tc1_specs,
    out_specs=_tc1_ospec,
    out_shape=jax.ShapeDtypeStruct((2, NP, D_HID), jnp.float32),
)


def _tc2_body(a0_ref, a1_ref, din0_ref, din1_ref, b1_ref, w2_ref, dout_ref,
              y2_ref):
    h = 0.5 * (a0_ref[...] * _scale(din0_ref[...])
               + a1_ref[...] * _scale(din1_ref[...])
               + b1_ref[0] + b1_ref[1])
    h = jnp.maximum(h, 0.0)
    y2 = jnp.dot(h, w2_ref[...], preferred_element_type=jnp.float32)
    y2_ref[...] = y2 * _scale(dout_ref[...])


_tc2_specs = [
    pl.BlockSpec((None, BM, D_HID), lambda r, i: (0, i, 0)),
    pl.BlockSpec((None, BM, D_HID), lambda r, i: (1, i, 0)),
    pl.BlockSpec((None, BM, 16), lambda r, i: (1, i, 0)),
    pl.BlockSpec((None, BM, 16), lambda r, i: (3, i, 0)),
    pl.BlockSpec((2, 1, D_HID), lambda r, i: (0, 0, 0)),
    pl.BlockSpec((None, D_HID, D_OUT), lambda r, i: (r, 0, 0)),
    pl.BlockSpec((None, BM, 16), lambda r, i: (2 * r, i, 0)),
]
_tc2_ospec = pl.BlockSpec((None, BM, D_OUT), lambda r, i: (r, i, 0))
_tc2 = pl.pallas_call(
    _tc2_body,
    grid=(2, GRID_M),
    in_specs=_tc2_specs,
    out_specs=_tc2_ospec,
    out_shape=jax.ShapeDtypeStruct((2, NP, D_OUT), jnp.float32),
)


def _tc3_body(a0_ref, a1_ref, din0_ref, din1_ref, b2_ref, wcat_ref, bcat_ref,
              out_ref):
    h2 = 0.5 * (a0_ref[...] * _scale(din0_ref[...])
                + a1_ref[...] * _scale(din1_ref[...])
                + b2_ref[0] + b2_ref[1])
    g = jnp.dot(h2, wcat_ref[...], preferred_element_type=jnp.float32)
    g = g + bcat_ref[...]
    sig = jax.nn.sigmoid
    hf = sig(g[:, 3 * H:4 * H]) * jnp.tanh(
        sig(g[:, 0:H]) * jnp.tanh(g[:, 2 * H:3 * H]))
    hb = sig(g[:, 7 * H:8 * H]) * jnp.tanh(
        sig(g[:, 4 * H:5 * H]) * jnp.tanh(g[:, 6 * H:7 * H]))
    out_ref[...] = jnp.concatenate([hf, hb], axis=1)


_tc3_specs = [
    pl.BlockSpec((None, BM, D_OUT), lambda i: (0, i, 0)),
    pl.BlockSpec((None, BM, D_OUT), lambda i: (1, i, 0)),
    pl.BlockSpec((None, BM, 16), lambda i: (1, i, 0)),
    pl.BlockSpec((None, BM, 16), lambda i: (3, i, 0)),
    pl.BlockSpec((2, 1, D_OUT), lambda i: (0, 0, 0)),
    pl.BlockSpec((D_OUT, 8 * H), lambda i: (0, 0)),
    pl.BlockSpec((1, 8 * H), lambda i: (0, 0)),
]
_tc3_ospec = pl.BlockSpec((BM, D_OUT), lambda i: (i, 0))
_tc3 = pl.pallas_call(
    _tc3_body,
    grid=(GRID_M,),
    in_specs=_tc3_specs,
    out_specs=_tc3_ospec,
    out_shape=jax.ShapeDtypeStruct((NP, D_OUT), jnp.float32),
)


def kernel(x, edge_index_rel0, edge_index_rel1, W1_0, b1_0, W1_1, b1_1,
           W2_0, b2_0, W2_1, b2_1, W_ih_f, W_hh_f, b_ih_f, b_hh_f,
           W_ih_r, W_hh_r, b_ih_r, b_hh_r):
    segpad = jnp.zeros(((NCHUNKP - NCHUNK) * CH,), jnp.int32)
    edges = jnp.concatenate([
        edge_index_rel0[0], segpad, edge_index_rel0[1], segpad,
        edge_index_rel1[0], segpad, edge_index_rel1[1], segpad,
    ])
    edges2d = edges.reshape(-1, CH)
    ones16 = jnp.ones((CH, 16), jnp.float32)
    zeros16 = jnp.zeros((NP, 16), jnp.float32)
    degs = _deg_kernel(edges2d, ones16, zeros16).reshape(4, NP, 16)

    x_pad = jnp.pad(x, ((0, NP - N), (0, 0)))
    w1s = jnp.stack([W1_0, W1_1])
    y1 = _tc1(x_pad, w1s, degs)

    zeros128 = jnp.zeros((NP, D_HID), jnp.float32)
    agg1 = _agg128(y1.reshape(2 * NP, D_HID), edges2d, zeros128)
    agg1 = agg1.reshape(2, NP, D_HID)

    b1s = jnp.stack([b1_0, b1_1]).reshape(2, 1, D_HID)
    w2s = jnp.stack([W2_0, W2_1])
    y2 = _tc2(agg1, agg1, degs, degs, b1s, w2s, degs)

    zeros64 = jnp.zeros((NP, D_OUT), jnp.float32)
    agg2 = _agg64(y2.reshape(2 * NP, D_OUT), edges2d, zeros64)
    agg2 = agg2.reshape(2, NP, D_OUT)

    b2s = jnp.stack([b2_0, b2_1]).reshape(2, 1, D_OUT)
    wcat = jnp.concatenate([W_ih_f, W_ih_r], axis=0).T
    bcat = (jnp.concatenate([b_ih_f + b_hh_f, b_ih_r + b_hh_r])
            .reshape(1, 8 * H))
    out = _tc3(agg2, agg2, degs, degs, b2s, wcat, bcat)
    return out[:N]


# speedup vs baseline: 8.2313x; 1.0211x over previous
import functools

import jax
import jax.numpy as jnp
from jax import lax
from jax.experimental import pallas as pl
from jax.experimental.pallas import tpu as pltpu
from jax.experimental.pallas import tpu_sc as plsc

N = 10000
NP = 10240
D_IN = 128
D_HID = 128
D_OUT = 64
E = 160000
H = D_OUT // 2

CH = 128
NCHUNK = E // CH
NSUB = 16
ITERS = -(-NCHUNK // NSUB)
RSTRIPE = NP // NSUB

_mesh = plsc.VectorSubcoreMesh(core_axis_name="c", subcore_axis_name="s",
                               num_cores=2, num_subcores=NSUB)


GROUP = 8
CPT = (-(-NCHUNK // NSUB) + GROUP - 1) // GROUP * GROUP
NGRP = CPT // GROUP
NCHUNKP = -(-NCHUNK // GROUP) * GROUP


def _deg_body(edges2d, ones_hbm, zeros16, degs_out,
              acc0, acc1, ones_v, idx_s, idx_d, sem_a, sem_b):
    c = lax.axis_index("c")
    sid = lax.axis_index("s")
    r0 = sid * RSTRIPE
    pltpu.sync_copy(zeros16.at[pl.ds(r0, RSTRIPE)], acc0.at[pl.ds(r0, RSTRIPE)])
    pltpu.sync_copy(zeros16.at[pl.ds(r0, RSTRIPE)], acc1.at[pl.ds(r0, RSTRIPE)])
    pltpu.sync_copy(ones_hbm, ones_v)
    plsc.subcore_barrier()

    lo = sid * CPT
    srow = (2 * c) * NCHUNKP
    drow = (2 * c + 1) * NCHUNKP

    def grp(g, carry):
        base = lo + g * GROUP

        @pl.when(base < NCHUNK)
        def _():
            pltpu.sync_copy(edges2d.at[pl.ds(srow + base, GROUP)], idx_s)
            pltpu.sync_copy(edges2d.at[pl.ds(drow + base, GROUP)], idx_d)
            for j in range(GROUP):
                ok = ((g * GROUP + j) < CPT) & ((base + j) < NCHUNK)

                @pl.when(ok)
                def _(j=j):
                    pltpu.async_copy(ones_v, acc0.at[idx_s.at[j]], sem_a,
                                     add=True)
                    pltpu.async_copy(ones_v, acc1.at[idx_d.at[j]], sem_b,
                                     add=True)
            for j in range(GROUP):
                ok = ((g * GROUP + j) < CPT) & ((base + j) < NCHUNK)

                @pl.when(ok)
                def _(j=j):
                    pltpu.make_async_copy(ones_v, acc0.at[idx_s.at[j]],
                                          sem_a).wait()
                    pltpu.make_async_copy(ones_v, acc1.at[idx_d.at[j]],
                                          sem_b).wait()

        return carry

    lax.fori_loop(0, NGRP, grp, 0)

    plsc.subcore_barrier()
    pltpu.sync_copy(acc0.at[pl.ds(r0, RSTRIPE)],
                    degs_out.at[pl.ds((2 * c) * NP + r0, RSTRIPE)])
    pltpu.sync_copy(acc1.at[pl.ds(r0, RSTRIPE)],
                    degs_out.at[pl.ds((2 * c + 1) * NP + r0, RSTRIPE)])


_deg_kernel = pl.kernel(
    _deg_body,
    out_type=jax.ShapeDtypeStruct((4 * NP, 16), jnp.float32),
    mesh=_mesh,
    compiler_params=pltpu.CompilerParams(use_tc_tiling_on_sc=False),
    scratch_types=[
        pltpu.VMEM_SHARED((NP, 16), jnp.float32),
        pltpu.VMEM_SHARED((NP, 16), jnp.float32),
        pltpu.VMEM((CH, 16), jnp.float32),
        pltpu.VMEM((GROUP, CH), jnp.int32),
        pltpu.VMEM((GROUP, CH), jnp.int32),
        pltpu.SemaphoreType.DMA,
        pltpu.SemaphoreType.DMA,
    ],
)


def _agg_body_2buf(y_flat, edges2d, zerosd, agg_out, acc,
              rows0, rows1, idx_s, idx_d, sem0, sem1):
    c = lax.axis_index("c")
    sid = lax.axis_index("s")
    r0 = sid * RSTRIPE
    pltpu.sync_copy(zerosd.at[pl.ds(r0, RSTRIPE)], acc.at[pl.ds(r0, RSTRIPE)])
    plsc.subcore_barrier()

    srow = (2 * c) * NCHUNKP
    drow = (2 * c + 1) * NCHUNKP
    row_off = c * NP
    lo = sid * CPT
    rowbufs = (rows0, rows1)
    sems = (sem0, sem1)

    def grp(g, carry):
        base = lo + g * GROUP

        @pl.when(base < NCHUNK)
        def _():
            pltpu.sync_copy(edges2d.at[pl.ds(srow + base, GROUP)], idx_s)
            pltpu.sync_copy(edges2d.at[pl.ds(drow + base, GROUP)], idx_d)
            for jj in range(GROUP):
                for k in range(CH // 16):
                    sl = pl.ds(k * 16, 16)
                    idx_s[jj, sl] = idx_s[jj, sl] + row_off

            def ok(j):
                return ((g * GROUP + j) < CPT) & ((base + j) < NCHUNK)

            @pl.when(ok(0))
            def _():
                pltpu.async_copy(y_flat.at[idx_s.at[0]], rows0, sem0)

            for j in range(GROUP):
                rows, sem = rowbufs[j % 2], sems[j % 2]
                if j + 1 < GROUP:
                    nrows, nsem = rowbufs[(j + 1) % 2], sems[(j + 1) % 2]

                    @pl.when(ok(j + 1))
                    def _(j=j, nrows=nrows, nsem=nsem):
                        pltpu.async_copy(y_flat.at[idx_s.at[j + 1]], nrows,
                                         nsem)

                @pl.when(ok(j))
                def _(j=j, rows=rows, sem=sem):
                    pltpu.make_async_copy(y_flat.at[idx_s.at[j]], rows,
                                          sem).wait()
                    pltpu.sync_copy(rows, acc.at[idx_d.at[j]], add=True)

        return carry

    lax.fori_loop(0, NGRP, grp, 0)

    plsc.subcore_barrier()
    pltpu.sync_copy(acc.at[pl.ds(r0, RSTRIPE)],
                    agg_out.at[pl.ds(row_off + r0, RSTRIPE)])


def _agg_body_4buf(y_flat, edges2d, zerosd, agg_out, acc,
              rows0, rows1, rows2, rows3, idx_s, idx_d,
              gsem0, gsem1, gsem2, gsem3, ssem0, ssem1, ssem2, ssem3):
    c = lax.axis_index("c")
    sid = lax.axis_index("s")
    r0 = sid * RSTRIPE
    pltpu.sync_copy(zerosd.at[pl.ds(r0, RSTRIPE)], acc.at[pl.ds(r0, RSTRIPE)])
    plsc.subcore_barrier()

    srow = (2 * c) * NCHUNKP
    drow = (2 * c + 1) * NCHUNKP
    row_off = c * NP
    lo = sid * CPT
    rowbufs = (rows0, rows1, rows2, rows3)
    gsems = (gsem0, gsem1, gsem2, gsem3)
    ssems = (ssem0, ssem1, ssem2, ssem3)

    def grp(g, carry):
        base = lo + g * GROUP

        @pl.when(base < NCHUNK)
        def _():
            pltpu.sync_copy(edges2d.at[pl.ds(srow + base, GROUP)], idx_s)
            pltpu.sync_copy(edges2d.at[pl.ds(drow + base, GROUP)], idx_d)
            for jj in range(GROUP):
                for k in range(CH // 16):
                    sl = pl.ds(k * 16, 16)
                    idx_s[jj, sl] = idx_s[jj, sl] + row_off

            def ok(j):
                return ((g * GROUP + j) < CPT) & ((base + j) < NCHUNK)

            def fire_gather(j):
                @pl.when(ok(j))
                def _():
                    pltpu.async_copy(y_flat.at[idx_s.at[j]], rowbufs[j % 4],
                                     gsems[j % 4])

            def wait_scatter(j):
                @pl.when(ok(j))
                def _():
                    pltpu.make_async_copy(rowbufs[j % 4],
                                          acc.at[idx_d.at[j]],
                                          ssems[j % 4]).wait()

            fire_gather(0)
            fire_gather(1)
            for j in range(GROUP):
                if j + 2 < GROUP:
                    if j - 2 >= 0:
                        wait_scatter(j - 2)
                    fire_gather(j + 2)

                @pl.when(ok(j))
                def _(j=j):
                    pltpu.make_async_copy(y_flat.at[idx_s.at[j]],
                                          rowbufs[j % 4], gsems[j % 4]).wait()
                    pltpu.async_copy(rowbufs[j % 4], acc.at[idx_d.at[j]],
                                     ssems[j % 4], add=True)

            for j in range(GROUP - 4, GROUP):
                wait_scatter(j)

        return carry

    lax.fori_loop(0, NGRP, grp, 0)

    plsc.subcore_barrier()
    pltpu.sync_copy(acc.at[pl.ds(r0, RSTRIPE)],
                    agg_out.at[pl.ds(row_off + r0, RSTRIPE)])


def _make_agg_kernel(d):
    nbuf = 2 if d == 128 else 4
    body = _agg_body_2buf if nbuf == 2 else _agg_body_4buf
    return pl.kernel(
        body,
        out_type=jax.ShapeDtypeStruct((2 * NP, d), jnp.float32),
        mesh=_mesh,
        compiler_params=pltpu.CompilerParams(use_tc_tiling_on_sc=(d == 128)),
        scratch_types=(
            [pltpu.VMEM_SHARED((NP, d), jnp.float32)]
            + [pltpu.VMEM((CH, d), jnp.float32)] * nbuf
            + [pltpu.VMEM((GROUP, CH), jnp.int32)] * 2
            + [pltpu.SemaphoreType.DMA] * (2 if nbuf == 2 else 8)
        ),
    )


_agg128 = _make_agg_kernel(D_HID)
_agg64 = _make_agg_kernel(D_OUT)


BM = 1024
GRID_M = NP // BM


def _scale(deg_blk):
    return lax.rsqrt(jnp.maximum(deg_blk[:, :1], 1.0))


def _tc1_body(x_ref, w_ref, dout_ref, y_ref):
    y = jnp.dot(x_ref[...], w_ref[...], preferred_element_type=jnp.float32)
    y_ref[...] = y * _scale(dout_ref[...])


_tc1_specs = [
    pl.BlockSpec((BM, D_IN), lambda r, i: (i, 0)),
    pl.BlockSpec((None, D_IN, D_HID), lambda r, i: (r, 0, 0)),
    pl.BlockSpec((None, BM, 16), lambda r, i: (2 * r, i, 0)),
]
_tc1_ospec = pl.BlockSpec((None, BM, D_HID), lambda r, i: (r, i, 0))
_tc1 = pl.pallas_call(
    _tc1_body,
    grid=(2, GRID_M),
    in_specs=_tc1_specs,
    out_specs=_tc1_ospec,
    out_shape=jax.ShapeDtypeStruct((2, NP, D_HID), jnp.float32),
)


def _tc2_body(a0_ref, a1_ref, din0_ref, din1_ref, b1_ref, w2_ref, dout_ref,
              y2_ref):
    h = 0.5 * (a0_ref[...] * _scale(din0_ref[...])
               + a1_ref[...] * _scale(din1_ref[...])
               + b1_ref[0] + b1_ref[1])
    h = jnp.maximum(h, 0.0)
    y2 = jnp.dot(h, w2_ref[...], preferred_element_type=jnp.float32)
    y2_ref[...] = y2 * _scale(dout_ref[...])


_tc2_specs = [
    pl.BlockSpec((None, BM, D_HID), lambda r, i: (0, i, 0)),
    pl.BlockSpec((None, BM, D_HID), lambda r, i: (1, i, 0)),
    pl.BlockSpec((None, BM, 16), lambda r, i: (1, i, 0)),
    pl.BlockSpec((None, BM, 16), lambda r, i: (3, i, 0)),
    pl.BlockSpec((2, 1, D_HID), lambda r, i: (0, 0, 0)),
    pl.BlockSpec((None, D_HID, D_OUT), lambda r, i: (r, 0, 0)),
    pl.BlockSpec((None, BM, 16), lambda r, i: (2 * r, i, 0)),
]
_tc2_ospec = pl.BlockSpec((None, BM, D_OUT), lambda r, i: (r, i, 0))
_tc2 = pl.pallas_call(
    _tc2_body,
    grid=(2, GRID_M),
    in_specs=_tc2_specs,
    out_specs=_tc2_ospec,
    out_shape=jax.ShapeDtypeStruct((2, NP, D_OUT), jnp.float32),
)


def _tc3_body(a0_ref, a1_ref, din0_ref, din1_ref, b2_ref, wcat_ref, bcat_ref,
              out_ref):
    h2 = 0.5 * (a0_ref[...] * _scale(din0_ref[...])
                + a1_ref[...] * _scale(din1_ref[...])
                + b2_ref[0] + b2_ref[1])
    g = jnp.dot(h2, wcat_ref[...], preferred_element_type=jnp.float32)
    g = g + bcat_ref[...]
    sig = jax.nn.sigmoid
    hf = sig(g[:, 3 * H:4 * H]) * jnp.tanh(
        sig(g[:, 0:H]) * jnp.tanh(g[:, 2 * H:3 * H]))
    hb = sig(g[:, 7 * H:8 * H]) * jnp.tanh(
        sig(g[:, 4 * H:5 * H]) * jnp.tanh(g[:, 6 * H:7 * H]))
    out_ref[...] = jnp.concatenate([hf, hb], axis=1)


_tc3_specs = [
    pl.BlockSpec((None, BM, D_OUT), lambda i: (0, i, 0)),
    pl.BlockSpec((None, BM, D_OUT), lambda i: (1, i, 0)),
    pl.BlockSpec((None, BM, 16), lambda i: (1, i, 0)),
    pl.BlockSpec((None, BM, 16), lambda i: (3, i, 0)),
    pl.BlockSpec((2, 1, D_OUT), lambda i: (0, 0, 0)),
    pl.BlockSpec((D_OUT, 8 * H), lambda i: (0, 0)),
    pl.BlockSpec((1, 8 * H), lambda i: (0, 0)),
]
_tc3_ospec = pl.BlockSpec((BM, D_OUT), lambda i: (i, 0))
_tc3 = pl.pallas_call(
    _tc3_body,
    grid=(GRID_M,),
    in_specs=_tc3_specs,
    out_specs=_tc3_ospec,
    out_shape=jax.ShapeDtypeStruct((NP, D_OUT), jnp.float32),
)


def kernel(x, edge_index_rel0, edge_index_rel1, W1_0, b1_0, W1_1, b1_1,
           W2_0, b2_0, W2_1, b2_1, W_ih_f, W_hh_f, b_ih_f, b_hh_f,
           W_ih_r, W_hh_r, b_ih_r, b_hh_r):
    segpad = jnp.zeros(((NCHUNKP - NCHUNK) * CH,), jnp.int32)
    edges = jnp.concatenate([
        edge_index_rel0[0], segpad, edge_index_rel0[1], segpad,
        edge_index_rel1[0], segpad, edge_index_rel1[1], segpad,
    ])
    edges2d = edges.reshape(-1, CH)
    ones16 = jnp.ones((CH, 16), jnp.float32)
    zeros16 = jnp.zeros((NP, 16), jnp.float32)
    degs = _deg_kernel(edges2d, ones16, zeros16).reshape(4, NP, 16)

    x_pad = jnp.pad(x, ((0, NP - N), (0, 0)))
    w1s = jnp.stack([W1_0, W1_1])
    y1 = _tc1(x_pad, w1s, degs)

    zeros128 = jnp.zeros((NP, D_HID), jnp.float32)
    agg1 = _agg128(y1.reshape(2 * NP, D_HID), edges2d, zeros128)
    agg1 = agg1.reshape(2, NP, D_HID)

    b1s = jnp.stack([b1_0, b1_1]).reshape(2, 1, D_HID)
    w2s = jnp.stack([W2_0, W2_1])
    y2 = _tc2(agg1, agg1, degs, degs, b1s, w2s, degs)

    zeros64 = jnp.zeros((NP, D_OUT), jnp.float32)
    agg2 = _agg64(y2.reshape(2 * NP, D_OUT), edges2d, zeros64)
    agg2 = agg2.reshape(2, NP, D_OUT)

    b2s = jnp.stack([b2_0, b2_1]).reshape(2, 1, D_OUT)
    wcat = jnp.concatenate([W_ih_f, W_ih_r], axis=0).T
    bcat = (jnp.concatenate([b_ih_f + b_hh_f, b_ih_r + b_hh_r])
            .reshape(1, 8 * H))
    out = _tc3(agg2, agg2, degs, degs, b2s, wcat, bcat)
    return out[:N]
